# baseline (speedup 1.0000x reference)
import functools

import jax
import jax.numpy as jnp
from jax import lax
from jax.experimental import pallas as pl
from jax.experimental.pallas import tpu as pltpu
from jax.experimental.pallas import tpu_sc as plsc

_EMB_DIM = 64
_LANES = 16
_CHUNK = 128
_B_POS = 16384
_B_NEG = 81920


def _row_dots(u_rows, v_rows, scores, n_rows, v_alt=None, alt_mask=None):
    for g in range(n_rows // _LANES):
        rows = lax.iota(jnp.int32, _LANES) + (g * _LANES)

        def d_body(d, acc):
            cols = jnp.full((_LANES,), 0, jnp.int32) + d
            u = plsc.load_gather(u_rows, [rows, cols])
            v = plsc.load_gather(v_rows, [rows, cols])
            if v_alt is not None:
                va = plsc.load_gather(v_alt, [rows, cols])
                v = jnp.where(alt_mask, va, v)
            return acc + u * v

        acc = lax.fori_loop(0, _EMB_DIM, d_body,
                            jnp.zeros((_LANES,), jnp.float32), unroll=8)
        scores[pl.ds(g * _LANES, _LANES)] = acc


def _make_sc_dots():
    info = plsc.get_sparse_core_info()
    nc, ns = info.num_cores, info.num_subcores
    nw = nc * ns
    pos_per_w = _B_POS // nw
    neg_per_w = _B_NEG // nw

    mesh = plsc.VectorSubcoreMesh(core_axis_name="c", subcore_axis_name="s")

    @functools.partial(
        pl.kernel,
        out_type=[
            jax.ShapeDtypeStruct((_B_POS,), jnp.float32),
            jax.ShapeDtypeStruct((_B_NEG,), jnp.float32),
        ],
        mesh=mesh,
        compiler_params=pltpu.CompilerParams(
            use_tc_tiling_on_sc=False, needs_layout_passes=False),
        scratch_types=[
            pltpu.VMEM((_CHUNK,), jnp.int32),
            pltpu.VMEM((_CHUNK,), jnp.int32),
            pltpu.VMEM((_CHUNK, _EMB_DIM), jnp.float32),
            pltpu.VMEM((_CHUNK, _EMB_DIM), jnp.float32),
            pltpu.VMEM((_CHUNK, _EMB_DIM), jnp.float32),
            pltpu.VMEM((_CHUNK,), jnp.float32),
            pltpu.VMEM((_LANES,), jnp.int32),
            pltpu.SemaphoreType.DMA,
        ],
    )
    def sc_dots(pos_u_hbm, pos_v_hbm, neg_u_hbm, neg_v_hbm, ord_hbm,
                u_hbm, v_hbm, pos_out, neg_out,
                idx_u, idx_v, u_rows, v_rows, v_alt, scores, ord_v, sem):
        wid = lax.axis_index("s") * nc + lax.axis_index("c")

        pltpu.sync_copy(ord_hbm, ord_v)
        alt_mask = ord_v[...] == 1

        def do_phase(iu_hbm, iv_hbm, n_per_w, out_hbm, is_pos):
            base_w = wid * n_per_w

            def chunk_body(c, _):
                base = base_w + c * _CHUNK
                pltpu.sync_copy(iu_hbm.at[pl.ds(base, _CHUNK)], idx_u)
                pltpu.sync_copy(iv_hbm.at[pl.ds(base, _CHUNK)], idx_v)
                pltpu.async_copy(u_hbm.at[idx_u], u_rows, sem).wait()
                pltpu.async_copy(v_hbm.at[idx_v], v_rows, sem).wait()
                if is_pos:
                    pltpu.async_copy(u_hbm.at[idx_v], v_alt, sem).wait()
                    _row_dots(u_rows, v_rows, scores, _CHUNK, v_alt, alt_mask)
                else:
                    _row_dots(u_rows, v_rows, scores, _CHUNK)
                pltpu.sync_copy(scores, out_hbm.at[pl.ds(base, _CHUNK)])
                return 0

            lax.fori_loop(0, n_per_w // _CHUNK, chunk_body, 0)

        do_phase(pos_u_hbm, pos_v_hbm, pos_per_w, pos_out, True)
        do_phase(neg_u_hbm, neg_v_hbm, neg_per_w, neg_out, False)

    return sc_dots


def _loss_body(pos_ref, neg_ref, out_ref):
    pos = pos_ref[...]
    neg = -neg_ref[...]
    ls_pos = jnp.minimum(pos, 0.0) - jnp.log1p(jnp.exp(-jnp.abs(pos)))
    ls_neg = jnp.minimum(neg, 0.0) - jnp.log1p(jnp.exp(-jnp.abs(neg)))
    out_ref[0, 0] = -(jnp.sum(ls_pos) + jnp.sum(ls_neg))


def kernel(pos_u, pos_v, neg_u, neg_v, order, U, V):
    ord_vec = jnp.full((_LANES,), order, dtype=jnp.int32)
    sc_dots = _make_sc_dots()
    pos_scores, neg_scores = sc_dots(
        pos_u.astype(jnp.int32), pos_v.astype(jnp.int32),
        neg_u.astype(jnp.int32), neg_v.astype(jnp.int32),
        ord_vec, U, V)

    loss = pl.pallas_call(
        _loss_body,
        out_shape=jax.ShapeDtypeStruct((1, 1), jnp.float32),
        out_specs=pl.BlockSpec(memory_space=pltpu.SMEM),
    )(pos_scores.reshape(128, 128), neg_scores.reshape(640, 128))
    return loss[0, 0]


# speedup vs baseline: 1.8565x; 1.8565x over previous
import functools

import jax
import jax.numpy as jnp
from jax import lax
from jax.experimental import pallas as pl
from jax.experimental.pallas import tpu as pltpu
from jax.experimental.pallas import tpu_sc as plsc

_EMB_DIM = 64
_TWIDTH = 2 * _EMB_DIM
_LANES = 16
_CHUNK = 128
_B_POS = 16384
_B_NEG = 81920


def _row_dots(u_rows, v_rows, scores, v_off):
    lanes = lax.iota(jnp.int32, _LANES)
    for g in range(_CHUNK // _LANES):
        partials = []
        for r in range(_LANES):
            row = g * _LANES + r
            acc = jnp.zeros((_LANES,), jnp.float32)
            for a in range(4):
                u = u_rows[row, pl.ds(a * 16, 16)]
                v = v_rows[row, pl.ds(v_off + a * 16, 16)]
                acc = acc + u * v
            partials.append(acc)
        dist = 8
        while len(partials) > 1:
            nxt = []
            m = (lanes & dist) == 0
            for i in range(0, len(partials), 2):
                a, b = partials[i], partials[i + 1]
                pa = jnp.take(a, lanes ^ dist)
                pb = jnp.take(b, lanes ^ dist)
                nxt.append(jnp.where(m, a, pb) + jnp.where(m, pa, b))
            partials = nxt
            dist //= 2
        scores[pl.ds(g * _LANES, _LANES)] = partials[0]


def _make_sc_dots():
    info = plsc.get_sparse_core_info()
    nc, ns = info.num_cores, info.num_subcores
    nw = nc * ns
    pos_per_w = _B_POS // nw
    neg_per_w = _B_NEG // nw
    tot_per_w = pos_per_w + neg_per_w

    mesh = plsc.VectorSubcoreMesh(core_axis_name="c", subcore_axis_name="s")

    @functools.partial(
        pl.kernel,
        out_type=[
            jax.ShapeDtypeStruct((_B_POS,), jnp.float32),
            jax.ShapeDtypeStruct((_B_NEG,), jnp.float32),
        ],
        mesh=mesh,
        compiler_params=pltpu.CompilerParams(needs_layout_passes=False),
        scratch_types=[
            pltpu.VMEM((tot_per_w,), jnp.int32),
            pltpu.VMEM((tot_per_w,), jnp.int32),
            pltpu.VMEM((_CHUNK, _TWIDTH), jnp.float32),
            pltpu.VMEM((_CHUNK, _TWIDTH), jnp.float32),
            pltpu.VMEM((_CHUNK, _TWIDTH), jnp.float32),
            pltpu.VMEM((_CHUNK, _TWIDTH), jnp.float32),
            pltpu.VMEM((_CHUNK,), jnp.float32),
            pltpu.VMEM((_LANES,), jnp.int32),
            pltpu.SemaphoreType.DMA,
            pltpu.SemaphoreType.DMA,
        ],
    )
    def sc_dots(pos_u_hbm, pos_v_hbm, neg_u_hbm, neg_v_hbm, ord_hbm,
                t_hbm, pos_out, neg_out,
                iu_all, iv_all, u0, v0, u1, v1, scores, ord_v, sem0, sem1):
        wid = lax.axis_index("s") * nc + lax.axis_index("c")

        pltpu.sync_copy(ord_hbm, ord_v)
        pos_off = jnp.where(jnp.max(ord_v[...]) == 1, 0, _EMB_DIM)
        neg_off = _EMB_DIM

        pltpu.sync_copy(pos_u_hbm.at[pl.ds(wid * pos_per_w, pos_per_w)],
                        iu_all.at[pl.ds(0, pos_per_w)])
        pltpu.sync_copy(pos_v_hbm.at[pl.ds(wid * pos_per_w, pos_per_w)],
                        iv_all.at[pl.ds(0, pos_per_w)])
        pltpu.sync_copy(neg_u_hbm.at[pl.ds(wid * neg_per_w, neg_per_w)],
                        iu_all.at[pl.ds(pos_per_w, neg_per_w)])
        pltpu.sync_copy(neg_v_hbm.at[pl.ds(wid * neg_per_w, neg_per_w)],
                        iv_all.at[pl.ds(pos_per_w, neg_per_w)])

        def do_phase(local_off, n_chunks, out_hbm, out_base, v_off):
            def issue(c, u_buf, v_buf, sem):
                off = local_off + c * _CHUNK
                pltpu.async_copy(t_hbm.at[iu_all.at[pl.ds(off, _CHUNK)]],
                                 u_buf, sem)
                pltpu.async_copy(t_hbm.at[iv_all.at[pl.ds(off, _CHUNK)]],
                                 v_buf, sem)

            def run(c, u_buf, v_buf, sem):
                iu = iu_all.at[pl.ds(local_off, _CHUNK)]
                pltpu.make_async_copy(t_hbm.at[iu], u_buf, sem).wait()
                pltpu.make_async_copy(t_hbm.at[iu], v_buf, sem).wait()
                _row_dots(u_buf, v_buf, scores, v_off)
                pltpu.sync_copy(
                    scores, out_hbm.at[pl.ds(out_base + c * _CHUNK, _CHUNK)])

            issue(0, u0, v0, sem0)

            def pair_body(p, _):
                c0 = 2 * p
                issue(c0 + 1, u1, v1, sem1)
                run(c0, u0, v0, sem0)

                @pl.when(c0 + 2 < n_chunks)
                def _():
                    issue(c0 + 2, u0, v0, sem0)

                run(c0 + 1, u1, v1, sem1)
                return 0

            lax.fori_loop(0, n_chunks // 2, pair_body, 0)

        do_phase(0, pos_per_w // _CHUNK, pos_out, wid * pos_per_w, pos_off)
        do_phase(pos_per_w, neg_per_w // _CHUNK, neg_out, wid * neg_per_w,
                 neg_off)

    return sc_dots


def _loss_body(pos_ref, neg_ref, out_ref):
    pos = pos_ref[...]
    neg = -neg_ref[...]
    ls_pos = jnp.minimum(pos, 0.0) - jnp.log1p(jnp.exp(-jnp.abs(pos)))
    ls_neg = jnp.minimum(neg, 0.0) - jnp.log1p(jnp.exp(-jnp.abs(neg)))
    out_ref[0, 0] = -(jnp.sum(ls_pos) + jnp.sum(ls_neg))


def kernel(pos_u, pos_v, neg_u, neg_v, order, U, V):
    ord_vec = jnp.full((_LANES,), order, dtype=jnp.int32)
    table = jnp.concatenate([U, V], axis=1)
    sc_dots = _make_sc_dots()
    pos_scores, neg_scores = sc_dots(
        pos_u.astype(jnp.int32), pos_v.astype(jnp.int32),
        neg_u.astype(jnp.int32), neg_v.astype(jnp.int32),
        ord_vec, table)

    loss = pl.pallas_call(
        _loss_body,
        out_shape=jax.ShapeDtypeStruct((1, 1), jnp.float32),
        out_specs=pl.BlockSpec(memory_space=pltpu.SMEM),
    )(pos_scores.reshape(128, 128), neg_scores.reshape(640, 128))
    return loss[0, 0]


# speedup vs baseline: 2.0117x; 1.0836x over previous
import functools

import jax
import jax.numpy as jnp
from jax import lax
from jax.experimental import pallas as pl
from jax.experimental.pallas import tpu as pltpu
from jax.experimental.pallas import tpu_sc as plsc

_EMB_DIM = 64
_LANES = 16
_CHUNK = 128
_B_POS = 16384
_B_NEG = 81920


def _row_dots(u_rows, v_rows, scores, s_base):
    lanes = lax.iota(jnp.int32, _LANES)
    for g in range(_CHUNK // _LANES):
        partials = []
        for r in range(_LANES):
            row = g * _LANES + r
            acc = jnp.zeros((_LANES,), jnp.float32)
            for a in range(4):
                u = u_rows[row, pl.ds(a * 16, 16)]
                v = v_rows[row, pl.ds(a * 16, 16)]
                acc = acc + u * v
            partials.append(acc)
        dist = 8
        while len(partials) > 1:
            nxt = []
            m = (lanes & dist) == 0
            for i in range(0, len(partials), 2):
                a, b = partials[i], partials[i + 1]
                pa = jnp.take(a, lanes ^ dist)
                pb = jnp.take(b, lanes ^ dist)
                nxt.append(jnp.where(m, a, pb) + jnp.where(m, pa, b))
            partials = nxt
            dist //= 2
        scores[pl.ds(s_base + g * _LANES, _LANES)] = partials[0]


def _make_sc_dots():
    info = plsc.get_sparse_core_info()
    nc, ns = info.num_cores, info.num_subcores
    nw = nc * ns
    pos_per_w = _B_POS // nw
    neg_per_w = _B_NEG // nw
    tot_per_w = pos_per_w + neg_per_w

    mesh = plsc.VectorSubcoreMesh(core_axis_name="c", subcore_axis_name="s")

    @functools.partial(
        pl.kernel,
        out_type=jax.ShapeDtypeStruct((nw, _LANES), jnp.float32),
        mesh=mesh,
        compiler_params=pltpu.CompilerParams(
            use_tc_tiling_on_sc=False, needs_layout_passes=False),
        scratch_types=[
            pltpu.VMEM((tot_per_w,), jnp.int32),
            pltpu.VMEM((tot_per_w,), jnp.int32),
            pltpu.VMEM((_CHUNK, _EMB_DIM), jnp.float32),
            pltpu.VMEM((_CHUNK, _EMB_DIM), jnp.float32),
            pltpu.VMEM((_CHUNK, _EMB_DIM), jnp.float32),
            pltpu.VMEM((_CHUNK, _EMB_DIM), jnp.float32),
            pltpu.VMEM((tot_per_w,), jnp.float32),
            pltpu.VMEM((_LANES,), jnp.int32),
            pltpu.SemaphoreType.DMA,
            pltpu.SemaphoreType.DMA,
        ],
    )
    def sc_dots(pos_u_hbm, pos_v_hbm, neg_u_hbm, neg_v_hbm, ord_hbm,
                u_hbm, v_hbm, part_out,
                iu_all, iv_all, u0, v0, u1, v1, scores, ord_v, sem0, sem1):
        wid = lax.axis_index("s") * nc + lax.axis_index("c")

        pltpu.sync_copy(ord_hbm, ord_v)
        is1 = jnp.max(ord_v[...]) == 1

        pltpu.sync_copy(pos_u_hbm.at[pl.ds(wid * pos_per_w, pos_per_w)],
                        iu_all.at[pl.ds(0, pos_per_w)])
        pltpu.sync_copy(pos_v_hbm.at[pl.ds(wid * pos_per_w, pos_per_w)],
                        iv_all.at[pl.ds(0, pos_per_w)])
        pltpu.sync_copy(neg_u_hbm.at[pl.ds(wid * neg_per_w, neg_per_w)],
                        iu_all.at[pl.ds(pos_per_w, neg_per_w)])
        pltpu.sync_copy(neg_v_hbm.at[pl.ds(wid * neg_per_w, neg_per_w)],
                        iv_all.at[pl.ds(pos_per_w, neg_per_w)])

        def do_phase(local_off, n_chunks, is_pos):
            def issue(c, u_buf, v_buf, sem):
                off = local_off + c * _CHUNK
                iu = iu_all.at[pl.ds(off, _CHUNK)]
                iv = iv_all.at[pl.ds(off, _CHUNK)]
                pltpu.async_copy(u_hbm.at[iu], u_buf, sem)
                if is_pos:
                    @pl.when(is1)
                    def _():
                        pltpu.async_copy(u_hbm.at[iv], v_buf, sem)

                    @pl.when(jnp.logical_not(is1))
                    def _():
                        pltpu.async_copy(v_hbm.at[iv], v_buf, sem)
                else:
                    pltpu.async_copy(v_hbm.at[iv], v_buf, sem)

            def run(c, u_buf, v_buf, sem):
                iu = iu_all.at[pl.ds(local_off, _CHUNK)]
                pltpu.make_async_copy(u_hbm.at[iu], u_buf, sem).wait()
                pltpu.make_async_copy(u_hbm.at[iu], v_buf, sem).wait()
                _row_dots(u_buf, v_buf, scores, local_off + c * _CHUNK)

            issue(0, u0, v0, sem0)

            def pair_body(p, _):
                c0 = 2 * p
                issue(c0 + 1, u1, v1, sem1)
                run(c0, u0, v0, sem0)

                @pl.when(c0 + 2 < n_chunks)
                def _():
                    issue(c0 + 2, u0, v0, sem0)

                run(c0 + 1, u1, v1, sem1)
                return 0

            lax.fori_loop(0, n_chunks // 2, pair_body, 0)

        do_phase(0, pos_per_w // _CHUNK, True)
        do_phase(pos_per_w, neg_per_w // _CHUNK, False)

        def logsig_sum(base, count, sign):
            def vec_body(i, acc):
                x = scores[pl.ds(base + i * _LANES, _LANES)] * sign
                w = 1.0 + jnp.exp(-jnp.abs(x))
                t = (w - 1.0) / (w + 1.0)
                t2 = t * t
                ln_w = 2.0 * t * (1.0 + t2 * (1.0 / 3.0 + t2 * (
                    1.0 / 5.0 + t2 * (1.0 / 7.0 + t2 * (1.0 / 9.0)))))
                return acc + (ln_w - jnp.minimum(x, 0.0))

            return lax.fori_loop(0, count // _LANES, vec_body,
                                 jnp.zeros((_LANES,), jnp.float32), unroll=4)

        part = logsig_sum(0, pos_per_w, 1.0) + logsig_sum(
            pos_per_w, neg_per_w, -1.0)
        scores[pl.ds(0, _LANES)] = part
        pltpu.sync_copy(scores.at[pl.ds(0, _LANES)], part_out.at[wid])

    return sc_dots


def kernel(pos_u, pos_v, neg_u, neg_v, order, U, V):
    ord_vec = jnp.full((_LANES,), order, dtype=jnp.int32)
    sc_dots = _make_sc_dots()
    partials = sc_dots(
        pos_u.astype(jnp.int32), pos_v.astype(jnp.int32),
        neg_u.astype(jnp.int32), neg_v.astype(jnp.int32),
        ord_vec, U, V)
    return jnp.sum(partials)


# speedup vs baseline: 2.4664x; 1.2261x over previous
import functools

import jax
import jax.numpy as jnp
from jax import lax
from jax.experimental import pallas as pl
from jax.experimental.pallas import tpu as pltpu
from jax.experimental.pallas import tpu_sc as plsc

_EMB_DIM = 64
_LANES = 16
_CHUNK = 256
_B_POS = 16384
_B_NEG = 81920


def _row_dots(u_rows, v_rows, scores, s_base):
    lanes = lax.iota(jnp.int32, _LANES)
    masks = [(lanes & d) == 0 for d in (8, 4, 2, 1)]
    perms = [lanes ^ d for d in (8, 4, 2, 1)]

    def combine(a, b, lvl):
        m = masks[lvl]
        pa = jnp.take(a, perms[lvl])
        pb = jnp.take(b, perms[lvl])
        return jnp.where(m, a, pb) + jnp.where(m, pa, b)

    def group_body(g, _):
        row0 = g * _LANES
        stack = []
        for r in range(_LANES):
            acc = jnp.zeros((_LANES,), jnp.float32)
            row = row0 + r
            for a in range(4):
                u = u_rows[row, pl.ds(a * 16, 16)]
                v = v_rows[row, pl.ds(a * 16, 16)]
                acc = acc + u * v
            node, lvl = acc, 0
            while stack and stack[-1][0] == lvl:
                _, prev = stack.pop()
                node = combine(prev, node, lvl)
                lvl += 1
            stack.append((lvl, node))
        scores[pl.ds(s_base + row0, _LANES)] = stack[0][1]
        return 0

    lax.fori_loop(0, _CHUNK // _LANES, group_body, 0)


def _make_sc_dots():
    info = plsc.get_sparse_core_info()
    nc, ns = info.num_cores, info.num_subcores
    nw = nc * ns
    pos_per_w = _B_POS // nw
    neg_per_w = _B_NEG // nw
    tot_per_w = pos_per_w + neg_per_w

    mesh = plsc.VectorSubcoreMesh(core_axis_name="c", subcore_axis_name="s")

    @functools.partial(
        pl.kernel,
        out_type=jax.ShapeDtypeStruct((nw, _LANES), jnp.float32),
        mesh=mesh,
        compiler_params=pltpu.CompilerParams(
            use_tc_tiling_on_sc=False, needs_layout_passes=False),
        scratch_types=[
            pltpu.VMEM((tot_per_w,), jnp.int32),
            pltpu.VMEM((tot_per_w,), jnp.int32),
            pltpu.VMEM((_CHUNK, _EMB_DIM), jnp.float32),
            pltpu.VMEM((_CHUNK, _EMB_DIM), jnp.float32),
            pltpu.VMEM((_CHUNK, _EMB_DIM), jnp.float32),
            pltpu.VMEM((_CHUNK, _EMB_DIM), jnp.float32),
            pltpu.VMEM((tot_per_w,), jnp.float32),
            pltpu.VMEM((_LANES,), jnp.int32),
            pltpu.SemaphoreType.DMA,
            pltpu.SemaphoreType.DMA,
        ],
    )
    def sc_dots(pos_u_hbm, pos_v_hbm, neg_u_hbm, neg_v_hbm, ord_hbm,
                u_hbm, v_hbm, part_out,
                iu_all, iv_all, u0, v0, u1, v1, scores, ord_v, sem0, sem1):
        wid = lax.axis_index("s") * nc + lax.axis_index("c")

        pltpu.sync_copy(ord_hbm, ord_v)
        is1 = jnp.max(ord_v[...]) == 1

        pltpu.sync_copy(pos_u_hbm.at[pl.ds(wid * pos_per_w, pos_per_w)],
                        iu_all.at[pl.ds(0, pos_per_w)])
        pltpu.sync_copy(pos_v_hbm.at[pl.ds(wid * pos_per_w, pos_per_w)],
                        iv_all.at[pl.ds(0, pos_per_w)])
        pltpu.sync_copy(neg_u_hbm.at[pl.ds(wid * neg_per_w, neg_per_w)],
                        iu_all.at[pl.ds(pos_per_w, neg_per_w)])
        pltpu.sync_copy(neg_v_hbm.at[pl.ds(wid * neg_per_w, neg_per_w)],
                        iv_all.at[pl.ds(pos_per_w, neg_per_w)])

        def do_phase(local_off, n_chunks, is_pos):
            def issue(c, u_buf, v_buf, sem):
                for h in range(2):
                    off = local_off + c * _CHUNK + h * 128
                    iu = iu_all.at[pl.ds(off, 128)]
                    iv = iv_all.at[pl.ds(off, 128)]
                    dst_u = u_buf.at[pl.ds(h * 128, 128)]
                    dst_v = v_buf.at[pl.ds(h * 128, 128)]
                    pltpu.async_copy(u_hbm.at[iu], dst_u, sem)
                    if is_pos:
                        @pl.when(is1)
                        def _():
                            pltpu.async_copy(u_hbm.at[iv], dst_v, sem)

                        @pl.when(jnp.logical_not(is1))
                        def _():
                            pltpu.async_copy(v_hbm.at[iv], dst_v, sem)
                    else:
                        pltpu.async_copy(v_hbm.at[iv], dst_v, sem)

            def run(c, u_buf, v_buf, sem):
                iu = iu_all.at[pl.ds(local_off, 128)]
                for h in range(2):
                    dst_u = u_buf.at[pl.ds(h * 128, 128)]
                    dst_v = v_buf.at[pl.ds(h * 128, 128)]
                    pltpu.make_async_copy(u_hbm.at[iu], dst_u, sem).wait()
                    pltpu.make_async_copy(u_hbm.at[iu], dst_v, sem).wait()
                _row_dots(u_buf, v_buf, scores, local_off + c * _CHUNK)

            issue(0, u0, v0, sem0)

            def pair_body(p, _):
                c0 = 2 * p
                issue(c0 + 1, u1, v1, sem1)
                run(c0, u0, v0, sem0)

                @pl.when(c0 + 2 < n_chunks)
                def _():
                    issue(c0 + 2, u0, v0, sem0)

                run(c0 + 1, u1, v1, sem1)
                return 0

            lax.fori_loop(0, n_chunks // 2, pair_body, 0)

        do_phase(0, pos_per_w // _CHUNK, True)
        do_phase(pos_per_w, neg_per_w // _CHUNK, False)

        def logsig_sum(base, count, sign):
            def vec_body(i, acc):
                x = scores[pl.ds(base + i * _LANES, _LANES)] * sign
                w = 1.0 + jnp.exp(-jnp.abs(x))
                t = (w - 1.0) / (w + 1.0)
                t2 = t * t
                ln_w = 2.0 * t * (1.0 + t2 * (1.0 / 3.0 + t2 * (
                    1.0 / 5.0 + t2 * (1.0 / 7.0 + t2 * (1.0 / 9.0)))))
                return acc + (ln_w - jnp.minimum(x, 0.0))

            return lax.fori_loop(0, count // _LANES, vec_body,
                                 jnp.zeros((_LANES,), jnp.float32), unroll=4)

        part = logsig_sum(0, pos_per_w, 1.0) + logsig_sum(
            pos_per_w, neg_per_w, -1.0)
        scores[pl.ds(0, _LANES)] = part
        pltpu.sync_copy(scores.at[pl.ds(0, _LANES)], part_out.at[wid])

    return sc_dots


def kernel(pos_u, pos_v, neg_u, neg_v, order, U, V):
    ord_vec = jnp.full((_LANES,), order, dtype=jnp.int32)
    sc_dots = _make_sc_dots()
    partials = sc_dots(
        pos_u.astype(jnp.int32), pos_v.astype(jnp.int32),
        neg_u.astype(jnp.int32), neg_v.astype(jnp.int32),
        ord_vec, U, V)
    return jnp.sum(partials)


# speedup vs baseline: 2.5334x; 1.0272x over previous
import functools

import jax
import jax.numpy as jnp
from jax import lax
from jax.experimental import pallas as pl
from jax.experimental.pallas import tpu as pltpu
from jax.experimental.pallas import tpu_sc as plsc

_EMB_DIM = 64
_LANES = 16
_CHUNK = 256
_B_POS = 16384
_B_NEG = 81920


def _row_dots(u_rows, v_rows, scores, s_base):
    lanes = lax.iota(jnp.int32, _LANES)
    masks = [(lanes & d) == 0 for d in (8, 4, 2, 1)]
    perms = [lanes ^ d for d in (8, 4, 2, 1)]

    def combine(a, b, lvl):
        m = masks[lvl]
        pa = jnp.take(a, perms[lvl])
        pb = jnp.take(b, perms[lvl])
        return jnp.where(m, a, pb) + jnp.where(m, pa, b)

    def group_body(g, _):
        row0 = g * _LANES
        stack = []
        for r in range(_LANES):
            acc = jnp.zeros((_LANES,), jnp.float32)
            row = row0 + r
            for a in range(4):
                u = u_rows[row, pl.ds(a * 16, 16)]
                v = v_rows[row, pl.ds(a * 16, 16)]
                acc = acc + u * v
            node, lvl = acc, 0
            while stack and stack[-1][0] == lvl:
                _, prev = stack.pop()
                node = combine(prev, node, lvl)
                lvl += 1
            stack.append((lvl, node))
        scores[pl.ds(s_base + row0, _LANES)] = stack[0][1]
        return 0

    lax.fori_loop(0, _CHUNK // _LANES, group_body, 0)


def _make_sc_dots():
    info = plsc.get_sparse_core_info()
    nc, ns = info.num_cores, info.num_subcores
    nw = nc * ns
    pos_per_w = _B_POS // nw
    neg_per_w = _B_NEG // nw
    tot_per_w = pos_per_w + neg_per_w

    mesh = plsc.VectorSubcoreMesh(core_axis_name="c", subcore_axis_name="s")

    @functools.partial(
        pl.kernel,
        out_type=jax.ShapeDtypeStruct((nw, _LANES), jnp.float32),
        mesh=mesh,
        compiler_params=pltpu.CompilerParams(
            use_tc_tiling_on_sc=False, needs_layout_passes=False),
        scratch_types=[
            pltpu.VMEM((tot_per_w,), jnp.int32),
            pltpu.VMEM((tot_per_w,), jnp.int32),
            pltpu.VMEM((_CHUNK, _EMB_DIM), jnp.float32),
            pltpu.VMEM((_CHUNK, _EMB_DIM), jnp.float32),
            pltpu.VMEM((_CHUNK, _EMB_DIM), jnp.float32),
            pltpu.VMEM((_CHUNK, _EMB_DIM), jnp.float32),
            pltpu.VMEM((_CHUNK, _EMB_DIM), jnp.float32),
            pltpu.VMEM((_CHUNK, _EMB_DIM), jnp.float32),
            pltpu.VMEM((tot_per_w,), jnp.float32),
            pltpu.VMEM((_LANES,), jnp.int32),
            pltpu.SemaphoreType.DMA,
            pltpu.SemaphoreType.DMA,
            pltpu.SemaphoreType.DMA,
        ],
    )
    def sc_dots(pos_u_hbm, pos_v_hbm, neg_u_hbm, neg_v_hbm, ord_hbm,
                u_hbm, v_hbm, part_out,
                iu_all, iv_all, u0, v0, u1, v1, u2, v2, scores, ord_v,
                sem0, sem1, sem2):
        wid = lax.axis_index("s") * nc + lax.axis_index("c")

        pltpu.sync_copy(ord_hbm, ord_v)
        is1 = jnp.max(ord_v[...]) == 1

        pltpu.sync_copy(pos_u_hbm.at[pl.ds(wid * pos_per_w, pos_per_w)],
                        iu_all.at[pl.ds(0, pos_per_w)])
        pltpu.sync_copy(pos_v_hbm.at[pl.ds(wid * pos_per_w, pos_per_w)],
                        iv_all.at[pl.ds(0, pos_per_w)])
        pltpu.sync_copy(neg_u_hbm.at[pl.ds(wid * neg_per_w, neg_per_w)],
                        iu_all.at[pl.ds(pos_per_w, neg_per_w)])
        pltpu.sync_copy(neg_v_hbm.at[pl.ds(wid * neg_per_w, neg_per_w)],
                        iv_all.at[pl.ds(pos_per_w, neg_per_w)])

        n_pos = pos_per_w // _CHUNK
        n_tot = tot_per_w // _CHUNK

        def issue(c, u_buf, v_buf, sem):
            for h in range(2):
                off = c * _CHUNK + h * 128
                iu = iu_all.at[pl.ds(off, 128)]
                iv = iv_all.at[pl.ds(off, 128)]
                dst_u = u_buf.at[pl.ds(h * 128, 128)]
                dst_v = v_buf.at[pl.ds(h * 128, 128)]
                pltpu.async_copy(u_hbm.at[iu], dst_u, sem)
                from_u = jnp.logical_and(c < n_pos, is1)

                @pl.when(from_u)
                def _():
                    pltpu.async_copy(u_hbm.at[iv], dst_v, sem)

                @pl.when(jnp.logical_not(from_u))
                def _():
                    pltpu.async_copy(v_hbm.at[iv], dst_v, sem)

        def run(c, u_buf, v_buf, sem):
            iu = iu_all.at[pl.ds(0, 128)]
            for h in range(2):
                dst_u = u_buf.at[pl.ds(h * 128, 128)]
                dst_v = v_buf.at[pl.ds(h * 128, 128)]
                pltpu.make_async_copy(u_hbm.at[iu], dst_u, sem).wait()
                pltpu.make_async_copy(u_hbm.at[iu], dst_v, sem).wait()
            _row_dots(u_buf, v_buf, scores, c * _CHUNK)

        issue(0, u0, v0, sem0)
        issue(1, u1, v1, sem1)

        def ring_body(p, _):
            c0 = 3 * p
            issue(c0 + 2, u2, v2, sem2)
            run(c0, u0, v0, sem0)

            @pl.when(c0 + 3 < n_tot)
            def _():
                issue(c0 + 3, u0, v0, sem0)

            run(c0 + 1, u1, v1, sem1)

            @pl.when(c0 + 4 < n_tot)
            def _():
                issue(c0 + 4, u1, v1, sem1)

            run(c0 + 2, u2, v2, sem2)
            return 0

        lax.fori_loop(0, n_tot // 3, ring_body, 0)

        def logsig_sum(base, count, sign):
            def vec_body(i, acc):
                x = scores[pl.ds(base + i * _LANES, _LANES)] * sign
                w = 1.0 + jnp.exp(-jnp.abs(x))
                t = (w - 1.0) / (w + 1.0)
                t2 = t * t
                ln_w = 2.0 * t * (1.0 + t2 * (1.0 / 3.0 + t2 * (
                    1.0 / 5.0 + t2 * (1.0 / 7.0 + t2 * (1.0 / 9.0)))))
                return acc + (ln_w - jnp.minimum(x, 0.0))

            return lax.fori_loop(0, count // _LANES, vec_body,
                                 jnp.zeros((_LANES,), jnp.float32), unroll=4)

        part = logsig_sum(0, pos_per_w, 1.0) + logsig_sum(
            pos_per_w, neg_per_w, -1.0)
        scores[pl.ds(0, _LANES)] = part
        pltpu.sync_copy(scores.at[pl.ds(0, _LANES)], part_out.at[wid])

    return sc_dots


def kernel(pos_u, pos_v, neg_u, neg_v, order, U, V):
    ord_vec = jnp.full((_LANES,), order, dtype=jnp.int32)
    sc_dots = _make_sc_dots()
    partials = sc_dots(
        pos_u.astype(jnp.int32), pos_v.astype(jnp.int32),
        neg_u.astype(jnp.int32), neg_v.astype(jnp.int32),
        ord_vec, U, V)
    return jnp.sum(partials)
